# baseline (speedup 1.0000x reference)
import functools

import jax
import jax.numpy as jnp
from jax import lax
from jax.experimental import pallas as pl
from jax.experimental.pallas import tpu as pltpu
from jax.experimental.pallas import tpu_sc as plsc

N = 10000
E = 320000
D = 128

NC = 2
NS = 16
NW = NC * NS

LANES = 16
CHUNK = 128
BATCH_E = 2 * CHUNK
NB_REAL = E // BATCH_E
NBT = 40
NB_PAD = NBT * NW + NW

NP = 10240
ROWS_PER_TILE = NP // NS


_SC_MESH = plsc.VectorSubcoreMesh(
    core_axis_name="c", subcore_axis_name="s",
    num_cores=NC, num_subcores=NS)


@functools.partial(
    pl.kernel,
    out_type=jax.ShapeDtypeStruct((NC, NP, D), jnp.float32),
    mesh=_SC_MESH,
    scratch_types=(
        pltpu.VMEM_SHARED((NP, D), jnp.float32),
        pltpu.VMEM((2, 4, CHUNK), jnp.int32),
        pltpu.VMEM((2, 2, CHUNK), jnp.float32),
        pltpu.VMEM((2 * CHUNK, D), jnp.float32),
        pltpu.SemaphoreType.DMA,
        pltpu.SemaphoreType.DMA,
        pltpu.SemaphoreType.DMA,
    ))
def _sc_agg(h_hbm, eidx_hbm, ew_hbm, acc_out,
            acc_sh, ibuf, ewbuf, rows, gsem, ssem, esem):
    c = lax.axis_index("c")
    s = lax.axis_index("s")
    tid = s * NC + c

    zeros16 = jnp.zeros((LANES,), jnp.float32)

    def _zrow(e, carry):
        for q in range(D // LANES):
            rows[e, pl.ds(q * LANES, LANES)] = zeros16
        return carry
    lax.fori_loop(0, 2 * CHUNK, _zrow, 0)

    row0 = s * ROWS_PER_TILE
    for p in range(ROWS_PER_TILE // CHUNK):
        pltpu.sync_copy(rows.at[pl.ds(0, CHUNK)],
                        acc_sh.at[pl.ds(row0 + p * CHUNK, CHUNK)])

    plsc.subcore_barrier()

    def _batch(i, carry):
        bi = tid + NW * i
        b = 0
        pltpu.sync_copy(eidx_hbm.at[bi], ibuf.at[b])
        pltpu.sync_copy(ew_hbm.at[bi], ewbuf.at[b])
        descs = [
            pltpu.async_copy(h_hbm.at[ibuf.at[b, j]],
                             rows.at[pl.ds(j * CHUNK, CHUNK)], gsem)
            for j in range(2)
        ]
        for dsc in descs:
            dsc.wait()
        for j in range(2):
            def _scale(g, cc):
                wv16 = ewbuf[b, j, pl.ds(g * LANES, LANES)]
                for l in range(LANES):
                    wval = wv16[l]
                    r = j * CHUNK + g * LANES + l
                    for q in range(D // LANES):
                        rows[r, pl.ds(q * LANES, LANES)] = (
                            rows[r, pl.ds(q * LANES, LANES)] * wval)
                return cc
            lax.fori_loop(0, CHUNK // LANES, _scale, 0)
        for j in range(2):
            pltpu.sync_copy(rows.at[pl.ds(j * CHUNK, CHUNK)],
                            acc_sh.at[ibuf.at[b, 2 + j]], add=True)
        return carry
    lax.fori_loop(0, NBT, _batch, 0)

    plsc.subcore_barrier()

    pltpu.sync_copy(acc_sh.at[pl.ds(row0, ROWS_PER_TILE)],
                    acc_out.at[c, pl.ds(row0, ROWS_PER_TILE)])


DEG_BATCH_ROWS = 4
DEG_NBATCH = E // (CHUNK * DEG_BATCH_ROWS)


@functools.partial(
    pl.kernel,
    out_type=jax.ShapeDtypeStruct((NC, NP, D), jnp.float32),
    mesh=_SC_MESH,
    scratch_types=(
        pltpu.VMEM_SHARED((NP, D), jnp.float32),
        pltpu.VMEM((DEG_BATCH_ROWS, CHUNK), jnp.int32),
        pltpu.VMEM((CHUNK, D), jnp.float32),
    ))
def _sc_deg(dst_hbm, deg_out, degt, idx_d, ones_buf):
    c = lax.axis_index("c")
    s = lax.axis_index("s")
    tid = s * NC + c

    zeros16 = jnp.zeros((LANES,), jnp.float32)
    ones16 = jnp.ones((LANES,), jnp.float32)
    row0 = s * ROWS_PER_TILE

    def _zob(i, carry):
        for q in range(D // LANES):
            ones_buf[i, pl.ds(q * LANES, LANES)] = zeros16
        return carry
    lax.fori_loop(0, CHUNK, _zob, 0)
    for p in range(ROWS_PER_TILE // CHUNK):
        pltpu.sync_copy(ones_buf, degt.at[pl.ds(row0 + p * CHUNK, CHUNK)])

    def _fob(i, carry):
        for q in range(D // LANES):
            ones_buf[i, pl.ds(q * LANES, LANES)] = ones16
        return carry
    lax.fori_loop(0, CHUNK, _fob, 0)

    plsc.subcore_barrier()

    nb = DEG_NBATCH // NW + jnp.where(tid < DEG_NBATCH % NW, 1, 0)

    def _batch(i, carry):
        brow = (tid + NW * i) * DEG_BATCH_ROWS
        pltpu.sync_copy(dst_hbm.at[pl.ds(brow, DEG_BATCH_ROWS)], idx_d)
        for j in range(DEG_BATCH_ROWS):
            pltpu.sync_copy(ones_buf, degt.at[idx_d.at[j]], add=True)
        return carry
    lax.fori_loop(0, nb, _batch, 0)

    plsc.subcore_barrier()

    pltpu.sync_copy(degt.at[pl.ds(row0, ROWS_PER_TILE)],
                    deg_out.at[c, pl.ds(row0, ROWS_PER_TILE)])


_TCR = 2048


def _tc1_body(x_ref, acc_ref, degp_ref, ws_ref, wn_ref, b_ref,
              h_ref, r_ref):
    deg = jnp.max(degp_ref[0] + degp_ref[1], axis=1)
    rec = 1.0 / jnp.maximum(deg, 1.0)
    agg = (acc_ref[0] + acc_ref[1]) * rec[:, None]
    dn = (((1,), (1,)), ((), ()))
    h = lax.dot_general(x_ref[...], ws_ref[...], dn,
                        preferred_element_type=jnp.float32)
    h = h + lax.dot_general(agg, wn_ref[...], dn,
                            preferred_element_type=jnp.float32)
    h = h + b_ref[...]
    h_ref[...] = jnp.maximum(h, 0.0)
    r_ref[...] = rec[None, :]


def _tc2_body(x_ref, acc_ref, r_ref, ws_ref, wn_ref, b_ref, o_ref):
    rec = r_ref[0]
    agg = (acc_ref[0] + acc_ref[1]) * rec[:, None]
    dn = (((1,), (1,)), ((), ()))
    h = lax.dot_general(x_ref[...], ws_ref[...], dn,
                        preferred_element_type=jnp.float32)
    h = h + lax.dot_general(agg, wn_ref[...], dn,
                            preferred_element_type=jnp.float32)
    o_ref[...] = h + b_ref[...]


def _tc_dense1(x, acc, degp, ws, wn, b2d):
    grid = (NP // _TCR,)
    return pl.pallas_call(
        _tc1_body,
        grid=grid,
        in_specs=[
            pl.BlockSpec((_TCR, D), lambda i: (i, 0)),
            pl.BlockSpec((NC, _TCR, D), lambda i: (0, i, 0)),
            pl.BlockSpec((NC, _TCR, D), lambda i: (0, i, 0)),
            pl.BlockSpec((D, D), lambda i: (0, 0)),
            pl.BlockSpec((D, D), lambda i: (0, 0)),
            pl.BlockSpec((1, D), lambda i: (0, 0)),
        ],
        out_specs=[
            pl.BlockSpec((_TCR, D), lambda i: (i, 0)),
            pl.BlockSpec((1, _TCR), lambda i: (0, i)),
        ],
        out_shape=[
            jax.ShapeDtypeStruct((N, D), jnp.float32),
            jax.ShapeDtypeStruct((1, NP), jnp.float32),
        ],
    )(x, acc, degp, ws, wn, b2d)


def _tc_dense2(x, acc, r2d, ws, wn, b2d):
    grid = (NP // _TCR,)
    return pl.pallas_call(
        _tc2_body,
        grid=grid,
        in_specs=[
            pl.BlockSpec((_TCR, D), lambda i: (i, 0)),
            pl.BlockSpec((NC, _TCR, D), lambda i: (0, i, 0)),
            pl.BlockSpec((1, _TCR), lambda i: (0, i)),
            pl.BlockSpec((D, D), lambda i: (0, 0)),
            pl.BlockSpec((D, D), lambda i: (0, 0)),
            pl.BlockSpec((1, D), lambda i: (0, 0)),
        ],
        out_specs=pl.BlockSpec((_TCR, D), lambda i: (i, 0)),
        out_shape=jax.ShapeDtypeStruct((N, D), jnp.float32),
    )(x, acc, r2d, ws, wn, b2d)


def kernel(in_feat, edge_index, weights, W1_self, W1_neigh, b1,
           W2_self, W2_neigh, b2):
    dst2d = edge_index[1].reshape(E // CHUNK, CHUNK)
    b1_2d = b1.reshape(1, D)
    b2_2d = b2.reshape(1, D)

    npad = NB_PAD - NB_REAL
    eidx = jnp.concatenate([edge_index[0].reshape(-1, 2, CHUNK),
                            edge_index[1].reshape(-1, 2, CHUNK)],
                           axis=1)
    eidx = jnp.pad(eidx, ((0, npad), (0, 0), (0, 0)))
    ew = jnp.pad(weights.reshape(-1, 2, CHUNK), ((0, npad), (0, 0), (0, 0)))

    degp = _sc_deg(dst2d)
    acc1 = _sc_agg(in_feat, eidx, ew)
    h1, r2d = _tc_dense1(in_feat, acc1, degp, W1_self, W1_neigh, b1_2d)
    acc2 = _sc_agg(h1, eidx, ew)
    out = _tc_dense2(h1, acc2, r2d, W2_self, W2_neigh, b2_2d)
    return out


# speedup vs baseline: 2.2268x; 2.2268x over previous
import functools

import jax
import jax.numpy as jnp
from jax import lax
from jax.experimental import pallas as pl
from jax.experimental.pallas import tpu as pltpu
from jax.experimental.pallas import tpu_sc as plsc

N = 10000
E = 320000
D = 128

NC = 2
NS = 16
NW = NC * NS

LANES = 16
CHUNK = 128
BATCH_E = 2 * CHUNK
NB_REAL = E // BATCH_E
NBT = 40
NB_PAD = NBT * NW + NW

NP = 10240
ROWS_PER_TILE = NP // NS


_SC_MESH = plsc.VectorSubcoreMesh(
    core_axis_name="c", subcore_axis_name="s",
    num_cores=NC, num_subcores=NS)


@functools.partial(
    pl.kernel,
    out_type=jax.ShapeDtypeStruct((NC, NP, D), jnp.float32),
    mesh=_SC_MESH,
    scratch_types=(
        pltpu.VMEM_SHARED((NP, D), jnp.float32),
        pltpu.VMEM((2, CHUNK), jnp.int32),
        pltpu.VMEM((2, CHUNK), jnp.int32),
        pltpu.VMEM((2, CHUNK), jnp.float32),
        pltpu.VMEM((2 * CHUNK, D), jnp.float32),
        pltpu.SemaphoreType.DMA,
        pltpu.SemaphoreType.DMA,
    ))
def _sc_agg(h_hbm, src_hbm, dst_hbm, w_hbm, acc_out,
            acc_sh, idx_s, idx_d, wbuf, rows, gsem, ssem):
    c = lax.axis_index("c")
    s = lax.axis_index("s")
    tid = s * NC + c

    zeros16 = jnp.zeros((LANES,), jnp.float32)

    def _zrow(e, carry):
        for q in range(D // LANES):
            rows[e, pl.ds(q * LANES, LANES)] = zeros16
        return carry
    lax.fori_loop(0, 2 * CHUNK, _zrow, 0)

    row0 = s * ROWS_PER_TILE
    for p in range(ROWS_PER_TILE // CHUNK):
        pltpu.sync_copy(rows.at[pl.ds(0, CHUNK)],
                        acc_sh.at[pl.ds(row0 + p * CHUNK, CHUNK)])

    plsc.subcore_barrier()

    pltpu.sync_copy(dst_hbm.at[pl.ds(2 * tid, 2)], idx_d)
    for j in range(2):
        pltpu.async_copy(rows.at[pl.ds(j * CHUNK, CHUNK)],
                         acc_sh.at[idx_d.at[j]], ssem,
                         add=True)

    nb = NB_REAL // NW + jnp.where(tid < NB_REAL % NW, 1, 0)

    def _batch(i, carry):
        brow = (tid + NW * i) * 2
        pltpu.sync_copy(src_hbm.at[pl.ds(brow, 2)], idx_s)
        pltpu.sync_copy(w_hbm.at[pl.ds(brow, 2)], wbuf)
        for j in range(2):
            pltpu.make_async_copy(rows.at[pl.ds(j * CHUNK, CHUNK)],
                                  acc_sh.at[pl.ds(0, CHUNK)],
                                  ssem).wait()
        descs = [
            pltpu.async_copy(h_hbm.at[idx_s.at[j]],
                             rows.at[pl.ds(j * CHUNK, CHUNK)], gsem)
            for j in range(2)
        ]
        pltpu.sync_copy(dst_hbm.at[pl.ds(brow, 2)], idx_d)
        for dsc in descs:
            dsc.wait()
        for j in range(2):
            def _scale(g, cc):
                wv16 = wbuf[j, pl.ds(g * LANES, LANES)]
                for l in range(LANES):
                    wval = wv16[l]
                    r = j * CHUNK + g * LANES + l
                    for q in range(D // LANES):
                        rows[r, pl.ds(q * LANES, LANES)] = (
                            rows[r, pl.ds(q * LANES, LANES)] * wval)
                return cc
            lax.fori_loop(0, CHUNK // LANES, _scale, 0)
        for j in range(2):
            pltpu.async_copy(rows.at[pl.ds(j * CHUNK, CHUNK)],
                             acc_sh.at[idx_d.at[j]], ssem, add=True)
        return carry
    lax.fori_loop(0, nb, _batch, 0)

    for j in range(2):
        pltpu.make_async_copy(rows.at[pl.ds(j * CHUNK, CHUNK)],
                              acc_sh.at[pl.ds(0, CHUNK)], ssem).wait()

    plsc.subcore_barrier()

    pltpu.sync_copy(acc_sh.at[pl.ds(row0, ROWS_PER_TILE)],
                    acc_out.at[c, pl.ds(row0, ROWS_PER_TILE)])


DEG_BATCH_ROWS = 4
DEG_NBATCH = E // (CHUNK * DEG_BATCH_ROWS)


@functools.partial(
    pl.kernel,
    out_type=jax.ShapeDtypeStruct((NC, NP, D), jnp.float32),
    mesh=_SC_MESH,
    scratch_types=(
        pltpu.VMEM_SHARED((NP, D), jnp.float32),
        pltpu.VMEM((DEG_BATCH_ROWS, CHUNK), jnp.int32),
        pltpu.VMEM((CHUNK, D), jnp.float32),
    ))
def _sc_deg(dst_hbm, deg_out, degt, idx_d, ones_buf):
    c = lax.axis_index("c")
    s = lax.axis_index("s")
    tid = s * NC + c

    zeros16 = jnp.zeros((LANES,), jnp.float32)
    ones16 = jnp.ones((LANES,), jnp.float32)
    row0 = s * ROWS_PER_TILE

    def _zob(i, carry):
        for q in range(D // LANES):
            ones_buf[i, pl.ds(q * LANES, LANES)] = zeros16
        return carry
    lax.fori_loop(0, CHUNK, _zob, 0)
    for p in range(ROWS_PER_TILE // CHUNK):
        pltpu.sync_copy(ones_buf, degt.at[pl.ds(row0 + p * CHUNK, CHUNK)])

    def _fob(i, carry):
        for q in range(D // LANES):
            ones_buf[i, pl.ds(q * LANES, LANES)] = ones16
        return carry
    lax.fori_loop(0, CHUNK, _fob, 0)

    plsc.subcore_barrier()

    nb = DEG_NBATCH // NW + jnp.where(tid < DEG_NBATCH % NW, 1, 0)

    def _batch(i, carry):
        brow = (tid + NW * i) * DEG_BATCH_ROWS
        pltpu.sync_copy(dst_hbm.at[pl.ds(brow, DEG_BATCH_ROWS)], idx_d)
        for j in range(DEG_BATCH_ROWS):
            pltpu.sync_copy(ones_buf, degt.at[idx_d.at[j]], add=True)
        return carry
    lax.fori_loop(0, nb, _batch, 0)

    plsc.subcore_barrier()

    pltpu.sync_copy(degt.at[pl.ds(row0, ROWS_PER_TILE)],
                    deg_out.at[c, pl.ds(row0, ROWS_PER_TILE)])


_TCR = 2048


def _tc1_body(x_ref, acc_ref, degp_ref, ws_ref, wn_ref, b_ref,
              h_ref, r_ref):
    deg = jnp.max(degp_ref[0] + degp_ref[1], axis=1)
    rec = 1.0 / jnp.maximum(deg, 1.0)
    agg = (acc_ref[0] + acc_ref[1]) * rec[:, None]
    dn = (((1,), (1,)), ((), ()))
    h = lax.dot_general(x_ref[...], ws_ref[...], dn,
                        preferred_element_type=jnp.float32)
    h = h + lax.dot_general(agg, wn_ref[...], dn,
                            preferred_element_type=jnp.float32)
    h = h + b_ref[...]
    h_ref[...] = jnp.maximum(h, 0.0)
    r_ref[...] = rec[None, :]


def _tc2_body(x_ref, acc_ref, r_ref, ws_ref, wn_ref, b_ref, o_ref):
    rec = r_ref[0]
    agg = (acc_ref[0] + acc_ref[1]) * rec[:, None]
    dn = (((1,), (1,)), ((), ()))
    h = lax.dot_general(x_ref[...], ws_ref[...], dn,
                        preferred_element_type=jnp.float32)
    h = h + lax.dot_general(agg, wn_ref[...], dn,
                            preferred_element_type=jnp.float32)
    o_ref[...] = h + b_ref[...]


def _tc_dense1(x, acc, degp, ws, wn, b2d):
    grid = (NP // _TCR,)
    return pl.pallas_call(
        _tc1_body,
        grid=grid,
        in_specs=[
            pl.BlockSpec((_TCR, D), lambda i: (i, 0)),
            pl.BlockSpec((NC, _TCR, D), lambda i: (0, i, 0)),
            pl.BlockSpec((NC, _TCR, D), lambda i: (0, i, 0)),
            pl.BlockSpec((D, D), lambda i: (0, 0)),
            pl.BlockSpec((D, D), lambda i: (0, 0)),
            pl.BlockSpec((1, D), lambda i: (0, 0)),
        ],
        out_specs=[
            pl.BlockSpec((_TCR, D), lambda i: (i, 0)),
            pl.BlockSpec((1, _TCR), lambda i: (0, i)),
        ],
        out_shape=[
            jax.ShapeDtypeStruct((N, D), jnp.float32),
            jax.ShapeDtypeStruct((1, NP), jnp.float32),
        ],
    )(x, acc, degp, ws, wn, b2d)


def _tc_dense2(x, acc, r2d, ws, wn, b2d):
    grid = (NP // _TCR,)
    return pl.pallas_call(
        _tc2_body,
        grid=grid,
        in_specs=[
            pl.BlockSpec((_TCR, D), lambda i: (i, 0)),
            pl.BlockSpec((NC, _TCR, D), lambda i: (0, i, 0)),
            pl.BlockSpec((1, _TCR), lambda i: (0, i)),
            pl.BlockSpec((D, D), lambda i: (0, 0)),
            pl.BlockSpec((D, D), lambda i: (0, 0)),
            pl.BlockSpec((1, D), lambda i: (0, 0)),
        ],
        out_specs=pl.BlockSpec((_TCR, D), lambda i: (i, 0)),
        out_shape=jax.ShapeDtypeStruct((N, D), jnp.float32),
    )(x, acc, r2d, ws, wn, b2d)


def kernel(in_feat, edge_index, weights, W1_self, W1_neigh, b1,
           W2_self, W2_neigh, b2):
    dst2d = edge_index[1].reshape(E // CHUNK, CHUNK)
    b1_2d = b1.reshape(1, D)
    b2_2d = b2.reshape(1, D)

    src2d = edge_index[0].reshape(E // CHUNK, CHUNK)
    w2d = weights.reshape(E // CHUNK, CHUNK)

    degp = _sc_deg(dst2d)
    acc1 = _sc_agg(in_feat, src2d, dst2d, w2d)
    h1, r2d = _tc_dense1(in_feat, acc1, degp, W1_self, W1_neigh, b1_2d)
    acc2 = _sc_agg(h1, src2d, dst2d, w2d)
    out = _tc_dense2(h1, acc2, r2d, W2_self, W2_neigh, b2_2d)
    return out
